# baseline (speedup 1.0000x reference)
import functools
import jax
import jax.numpy as jnp
from jax import lax
from jax.experimental import pallas as pl
from jax.experimental.pallas import tpu as pltpu, tpu_sc as plsc

_BATCH = 16384
_D = 1000
_NW = 32
_CPW = _BATCH // _NW
_CW = 512
_RW = 40
_NCC = _CPW // _CW
_NRC = _D // _RW
_L = 16


def _sc_body(x_hbm, out_hbm, idx_v, buf0, buf1, sem0, sem1):
    wid = lax.axis_index("s") * 2 + lax.axis_index("c")
    col_base = wid * _CPW
    pltpu.sync_copy(x_hbm.at[pl.ds(col_base, _CPW)], idx_v)

    zeros = jnp.zeros((_L,), jnp.float32)
    ones = jnp.ones((_L,), jnp.float32)
    lane = lax.iota(jnp.int32, _L)

    bufs = (buf0, buf1)
    sems = (sem0, sem1)

    def _zero_buf(buf):
        def body(r, _):
            for j in range(_CW // _L):
                buf[r, pl.ds(j * _L, _L)] = zeros
            return 0
        lax.fori_loop(0, _RW, body, 0)

    _zero_buf(buf0)
    _zero_buf(buf1)

    chunks = [(cc, rc) for cc in range(_NCC) for rc in range(_NRC)]

    def _scatter(buf, chunk_id, vals):
        cc, rc = chunks[chunk_id]
        r0 = rc * _RW
        def gbody(g, _):
            iv = idx_v[pl.ds(cc * _CW + g * _L, _L)]
            mask = (iv >= r0) & (iv < r0 + _RW)
            row = jnp.where(mask, iv - r0, 0)
            plsc.store_scatter(buf, [row, lane + g * _L], vals, mask=mask)
            return 0
        lax.fori_loop(0, _CW // _L, gbody, 0)

    copies = [None, None]
    for c in range(len(chunks)):
        b = c % 2
        buf = bufs[b]
        if c >= 2:
            copies[b].wait()
            _scatter(buf, c - 2, zeros)
        _scatter(buf, c, ones)
        cc, rc = chunks[c]
        cp = pltpu.make_async_copy(
            buf,
            out_hbm.at[pl.ds(rc * _RW, _RW), pl.ds(col_base + cc * _CW, _CW)],
            sems[b])
        cp.start()
        copies[b] = cp
    copies[(len(chunks) - 2) % 2].wait()
    copies[(len(chunks) - 1) % 2].wait()


@jax.jit
def _sc_onehot_t(x):
    mesh = plsc.VectorSubcoreMesh(core_axis_name="c", subcore_axis_name="s")
    f = pl.kernel(
        _sc_body,
        mesh=mesh,
        compiler_params=pltpu.CompilerParams(
            needs_layout_passes=False,
            use_tc_tiling_on_sc=True,
        ),
        out_type=jax.ShapeDtypeStruct((_D, _BATCH), jnp.float32),
        scratch_types=[
            pltpu.VMEM((_CPW,), jnp.int32),
            pltpu.VMEM((_RW, _CW), jnp.float32),
            pltpu.VMEM((_RW, _CW), jnp.float32),
            pltpu.SemaphoreType.DMA,
            pltpu.SemaphoreType.DMA,
        ],
    )
    return f(x)


def kernel(x, table):
    del table
    return _sc_onehot_t(x).T


# speedup vs baseline: 2.1330x; 2.1330x over previous
import jax
import jax.numpy as jnp
from jax.experimental import pallas as pl

_BATCH = 16384
_NUM_CLASS = 1000
_BN = 1024


def _onehot_t_block(x_ref, o_ref):
    xb = x_ref[0, 0, :]
    rows = jax.lax.broadcasted_iota(jnp.int32, o_ref.shape, 0)
    o_ref[...] = jnp.where(rows == xb[None, :], 1.0, 0.0).astype(o_ref.dtype)


def kernel(x, table):
    del table
    grid = _BATCH // _BN
    x3 = x.reshape(grid, 1, _BN)
    out_t = pl.pallas_call(
        _onehot_t_block,
        grid=(grid,),
        in_specs=[pl.BlockSpec((1, 1, _BN), lambda i: (i, 0, 0))],
        out_specs=pl.BlockSpec((_NUM_CLASS, _BN), lambda i: (0, i)),
        out_shape=jax.ShapeDtypeStruct((_NUM_CLASS, _BATCH), jnp.float32),
    )(x3)
    return out_t.T
